# Initial kernel scaffold; baseline (speedup 1.0000x reference)
import jax
import jax.numpy as jnp
from jax.experimental import pallas as pl


def kernel(frequency, selected_indices):
    raise NotImplementedError("write your pallas kernel here")


# speedup vs baseline: 2.1900x; 2.1900x over previous
import functools

import jax
import jax.numpy as jnp
from jax import lax
from jax.experimental import pallas as pl
from jax.experimental.pallas import tpu as pltpu
from jax.experimental.pallas import tpu_sc as plsc

N = 1_000_000
B = 16_384
NW = 25
CH = N // NW
L = 16


def _body(f_hbm, idx_hbm, out_hbm, chunk_v, idx_v, vals_v):
    c = lax.axis_index("c")
    s = lax.axis_index("s")
    wid = s * 2 + c

    @pl.when(wid < NW)
    def _():
        base = pl.multiple_of(wid * CH, 8)
        pltpu.sync_copy(f_hbm.at[pl.ds(base, CH)], chunk_v)
        pltpu.sync_copy(idx_hbm, idx_v)

        def pass1(i, carry):
            idx = idx_v[pl.ds(i * L, L)]
            rel = jnp.clip(idx - base, 0, CH - 1)
            g = plsc.load_gather(chunk_v, [rel])
            vals_v[pl.ds(i * L, L)] = g + 1.0
            return carry

        lax.fori_loop(0, B // L, pass1, 0)

        def pass2(i, carry):
            idx = idx_v[pl.ds(i * L, L)]
            rel = idx - base
            m = (rel >= 0) & (rel < CH)
            relc = jnp.clip(rel, 0, CH - 1)
            v = vals_v[pl.ds(i * L, L)]
            plsc.store_scatter(chunk_v, [relc], v, mask=m)
            return carry

        lax.fori_loop(0, B // L, pass2, 0)

        pltpu.sync_copy(chunk_v, out_hbm.at[pl.ds(base, CH)])


def kernel(frequency, selected_indices):
    mesh = plsc.VectorSubcoreMesh(core_axis_name="c", subcore_axis_name="s")
    k = functools.partial(
        pl.kernel,
        mesh=mesh,
        out_type=jax.ShapeDtypeStruct((N,), jnp.float32),
        compiler_params=pltpu.CompilerParams(needs_layout_passes=False),
        scratch_types=[
            pltpu.VMEM((CH,), jnp.float32),
            pltpu.VMEM((B,), jnp.int32),
            pltpu.VMEM((B,), jnp.float32),
        ],
    )(_body)
    return k(frequency, selected_indices)


# speedup vs baseline: 2.2510x; 1.0279x over previous
import functools

import jax
import jax.numpy as jnp
from jax import lax
from jax.experimental import pallas as pl
from jax.experimental.pallas import tpu as pltpu
from jax.experimental.pallas import tpu_sc as plsc

N = 1_000_000
B = 16_384
NW = 25
CH = N // NW
L = 16
U = 8


def _body(f_hbm, idx_hbm, out_hbm, orig_v, outc_v, idx_v, sem0, sem1, sem2):
    c = lax.axis_index("c")
    s = lax.axis_index("s")
    wid = s * 2 + c

    @pl.when(wid < NW)
    def _():
        base = pl.multiple_of(wid * CH, 8)
        cp0 = pltpu.async_copy(f_hbm.at[pl.ds(base, CH)], orig_v, sem0)
        cp1 = pltpu.async_copy(f_hbm.at[pl.ds(base, CH)], outc_v, sem1)
        cp2 = pltpu.async_copy(idx_hbm, idx_v, sem2)
        cp0.wait()
        cp1.wait()
        cp2.wait()

        def scan(j, carry):
            off = j * (L * U)
            for u in range(U):
                idx = idx_v[pl.ds(off + u * L, L)]
                rel = idx - base
                m = (rel >= 0) & (rel < CH)
                relc = jnp.clip(rel, 0, CH - 1)
                g = plsc.load_gather(orig_v, [relc])
                plsc.store_scatter(outc_v, [relc], g + 1.0, mask=m)
            return carry

        lax.fori_loop(0, B // (L * U), scan, 0)

        pltpu.sync_copy(outc_v, out_hbm.at[pl.ds(base, CH)])


def kernel(frequency, selected_indices):
    mesh = plsc.VectorSubcoreMesh(core_axis_name="c", subcore_axis_name="s")
    k = functools.partial(
        pl.kernel,
        mesh=mesh,
        out_type=jax.ShapeDtypeStruct((N,), jnp.float32),
        compiler_params=pltpu.CompilerParams(needs_layout_passes=False),
        scratch_types=[
            pltpu.VMEM((CH,), jnp.float32),
            pltpu.VMEM((CH,), jnp.float32),
            pltpu.VMEM((B,), jnp.int32),
            pltpu.SemaphoreType.DMA,
            pltpu.SemaphoreType.DMA,
            pltpu.SemaphoreType.DMA,
        ],
    )(_body)
    return k(frequency, selected_indices)
